# Initial kernel scaffold; baseline (speedup 1.0000x reference)
import jax
import jax.numpy as jnp
from jax.experimental import pallas as pl


def kernel(tensor, bank_selections, weight, bias):
    raise NotImplementedError("write your pallas kernel here")


# speedup vs baseline: 2.3508x; 2.3508x over previous
import jax
import jax.numpy as jnp
from jax.experimental import pallas as pl

IN_F = 768
OUT_F = 768
N_BANKS = 64
N_ROWS = 128


def _body(sel_ref, xt_ref, bt_ref, w_ref, out_ref):
    d = pl.program_id(0)
    sel = sel_ref[...]

    @pl.when(d == 0)
    def _init():
        onehot = (
            jax.lax.broadcasted_iota(jnp.int32, (N_BANKS, N_ROWS), 0) == sel
        ).astype(jnp.float32)
        out_ref[...] = jax.lax.dot_general(
            bt_ref[...], onehot, (((1,), (0,)), ((), ())),
            preferred_element_type=jnp.float32)

    w = w_ref[0].astype(jnp.bfloat16)
    x = xt_ref[...].astype(jnp.bfloat16)
    y = jax.lax.dot_general(
        w, x, (((1,), (0,)), ((), ())),
        preferred_element_type=jnp.float32)
    mask = sel == d
    out_ref[...] += jnp.where(mask, y, 0.0)


def kernel(tensor, bank_selections, weight, bias):
    xt = tensor.reshape(N_ROWS, IN_F).T
    bt = bias.T
    sel = bank_selections.reshape(1, N_ROWS).astype(jnp.int32)

    out_t = pl.pallas_call(
        _body,
        grid=(N_BANKS,),
        in_specs=[
            pl.BlockSpec((1, N_ROWS), lambda d: (0, 0)),
            pl.BlockSpec((IN_F, N_ROWS), lambda d: (0, 0)),
            pl.BlockSpec((OUT_F, N_BANKS), lambda d: (0, 0)),
            pl.BlockSpec((1, OUT_F, IN_F), lambda d: (d, 0, 0)),
        ],
        out_specs=pl.BlockSpec((OUT_F, N_ROWS), lambda d: (0, 0)),
        out_shape=jax.ShapeDtypeStruct((OUT_F, N_ROWS), jnp.float32),
    )(sel, xt, bt, weight)

    return out_t.T.reshape(tensor.shape[0], tensor.shape[1], OUT_F)


# speedup vs baseline: 2.3633x; 1.0053x over previous
import jax
import jax.numpy as jnp
from jax.experimental import pallas as pl

IN_F = 768
OUT_F = 768
N_BANKS = 64
N_ROWS = 128


def _body(sel_ref, xt_ref, bt_ref, w_ref, out_ref):
    d = pl.program_id(0)
    sel = sel_ref[...]

    @pl.when(d == 0)
    def _init():
        onehot = (
            jax.lax.broadcasted_iota(jnp.int32, (N_BANKS, N_ROWS), 0) == sel
        ).astype(jnp.float32)
        out_ref[...] = jax.lax.dot_general(
            bt_ref[...], onehot, (((1,), (0,)), ((), ())),
            preferred_element_type=jnp.float32)

    w = w_ref[0]
    x = xt_ref[...]
    y = jax.lax.dot_general(
        w, x, (((1,), (0,)), ((), ())),
        preferred_element_type=jnp.float32)
    mask = sel == d
    out_ref[...] += jnp.where(mask, y, 0.0)


def kernel(tensor, bank_selections, weight, bias):
    xt = tensor.reshape(N_ROWS, IN_F).T
    bt = bias.T
    sel = bank_selections.reshape(1, N_ROWS).astype(jnp.int32)

    out_t = pl.pallas_call(
        _body,
        grid=(N_BANKS,),
        in_specs=[
            pl.BlockSpec((1, N_ROWS), lambda d: (0, 0)),
            pl.BlockSpec((IN_F, N_ROWS), lambda d: (0, 0)),
            pl.BlockSpec((OUT_F, N_BANKS), lambda d: (0, 0)),
            pl.BlockSpec((1, OUT_F, IN_F), lambda d: (d, 0, 0)),
        ],
        out_specs=pl.BlockSpec((OUT_F, N_ROWS), lambda d: (0, 0)),
        out_shape=jax.ShapeDtypeStruct((OUT_F, N_ROWS), jnp.float32),
    )(sel, xt, bt, weight)

    return out_t.T.reshape(tensor.shape[0], tensor.shape[1], OUT_F)


# speedup vs baseline: 3.3250x; 1.4069x over previous
import jax
import jax.numpy as jnp
from jax.experimental import pallas as pl
from jax.experimental.pallas import tpu as pltpu

IN_F = 768
OUT_F = 768
N_BANKS = 64
N_ROWS = 128
NBUF = 8


def _body(uniq_ref, nd_ref, sel_ref, xt_ref, bt_ref, w_hbm, out_ref,
          wbuf, sems):
    nd = nd_ref[0]
    sel = sel_ref[...]

    def copy_in(i, slot):
        return pltpu.make_async_copy(
            w_hbm.at[uniq_ref[i]], wbuf.at[slot], sems.at[slot])

    for i in range(NBUF):
        @pl.when(i < nd)
        def _(i=i):
            copy_in(i, i).start()

    onehot = (
        jax.lax.broadcasted_iota(jnp.int32, (N_BANKS, N_ROWS), 0) == sel
    ).astype(jnp.float32)
    out_ref[...] = jax.lax.dot_general(
        bt_ref[...], onehot, (((1,), (0,)), ((), ())),
        preferred_element_type=jnp.float32)

    x = xt_ref[...]

    def step(i, carry):
        slot = jax.lax.rem(i, NBUF)
        copy_in(i, slot).wait()
        y = jax.lax.dot_general(
            wbuf[slot], x, (((1,), (0,)), ((), ())),
            preferred_element_type=jnp.float32)
        mask = sel == uniq_ref[i]
        out_ref[...] += jnp.where(mask, y, 0.0)

        @pl.when(i + NBUF < nd)
        def _():
            copy_in(i + NBUF, slot).start()
        return carry

    jax.lax.fori_loop(0, nd, step, 0)


def kernel(tensor, bank_selections, weight, bias):
    xt = tensor.reshape(N_ROWS, IN_F).T
    bt = bias.T
    flat = bank_selections.reshape(-1).astype(jnp.int32)
    s = jnp.sort(flat)
    is_new = jnp.concatenate([jnp.array([True]), s[1:] != s[:-1]])
    pos = jnp.cumsum(is_new) - 1
    uniq = jnp.full((N_BANKS,), s[-1], jnp.int32).at[pos].set(s)
    ndis = is_new.sum(dtype=jnp.int32).reshape(1)
    sel2d = flat.reshape(1, N_ROWS)

    out_t = pl.pallas_call(
        _body,
        in_specs=[
            pl.BlockSpec(memory_space=pltpu.SMEM),
            pl.BlockSpec(memory_space=pltpu.SMEM),
            pl.BlockSpec(memory_space=pltpu.VMEM),
            pl.BlockSpec(memory_space=pltpu.VMEM),
            pl.BlockSpec(memory_space=pltpu.VMEM),
            pl.BlockSpec(memory_space=pl.ANY),
        ],
        out_specs=pl.BlockSpec(memory_space=pltpu.VMEM),
        out_shape=jax.ShapeDtypeStruct((OUT_F, N_ROWS), jnp.float32),
        scratch_shapes=[
            pltpu.VMEM((NBUF, OUT_F, IN_F), jnp.float32),
            pltpu.SemaphoreType.DMA((NBUF,)),
        ],
    )(uniq, ndis, sel2d, xt, bt, weight)

    return out_t.T.reshape(tensor.shape[0], tensor.shape[1], OUT_F)
